# baseline (speedup 1.0000x reference)
import jax
import jax.numpy as jnp
from jax import lax
from jax.experimental import pallas as pl
from jax.experimental.pallas import tpu as pltpu
from jax.experimental.pallas import tpu_sc as plsc

N = 50000
D = 256
B = 64
CHUNK = 80
GROUP = 8
NGROUPS = CHUNK // GROUP
NCOL = D // 16
NCORES = 2
NSUBCORES = 16


def _lower_bound(batch_ref, t):
    def step(_, c):
        lo, hi = c
        mid = (lo + hi) // 2
        idx = jnp.full((16,), mid, jnp.int32)
        val = jnp.min(plsc.load_gather(batch_ref, [idx]))
        go_right = val < t
        live = lo < hi
        new_lo = jnp.where(live & go_right, mid + 1, lo)
        new_hi = jnp.where(live & ~go_right, mid, hi)
        return (new_lo, new_hi)

    lo, _ = lax.fori_loop(0, 16, step, (jnp.int32(0), jnp.int32(N)))
    return lo


def _tree_sum(vals):
    while len(vals) > 1:
        vals = [vals[i] + vals[i + 1] for i in range(0, len(vals) - 1, 2)] + (
            [vals[-1]] if len(vals) % 2 else [])
    return vals[0]


def _tree_max(vals):
    while len(vals) > 1:
        vals = [jnp.maximum(vals[i], vals[i + 1])
                for i in range(0, len(vals) - 1, 2)] + (
            [vals[-1]] if len(vals) % 2 else [])
    return vals[0]


def _accum_group(chunk_v, buf_row0, r0, lo, hi, acc, masked):
    zerov = jnp.zeros((16,), jnp.float32)
    ninfv = jnp.full((16,), -jnp.inf, jnp.float32)
    if masked:
        vvs = [jnp.full((16,), ((r0 + u) >= lo) & ((r0 + u) < hi))
               for u in range(GROUP)]
    new = list(acc)
    for c in range(NCOL):
        xv = [chunk_v[buf_row0 + r0 + u, pl.ds(c * 16, 16)]
              for u in range(GROUP)]
        if masked:
            xs = [jnp.where(vvs[u], xv[u], zerov) for u in range(GROUP)]
            xm = [jnp.where(vvs[u], xv[u], ninfv) for u in range(GROUP)]
        else:
            xs = xv
            xm = xv
        new[c] = new[c] + _tree_sum(xs)
        new[NCOL + c] = jnp.maximum(new[NCOL + c], _tree_max(xm))
    return tuple(new)


def _process_graph(x_hbm, chunk_v, row_v, mean_hbm, amax_hbm, sem, g, s, e):
    gout = pl.multiple_of(g * D, 8)

    @pl.when(s >= e)
    def _():
        zerov = jnp.zeros((16,), jnp.float32)
        for c in range(NCOL):
            row_v[pl.ds(c * 16, 16)] = zerov
            row_v[pl.ds(D + c * 16, 16)] = zerov
        pltpu.sync_copy(row_v.at[pl.ds(0, D)], mean_hbm.at[pl.ds(gout, D)])
        pltpu.sync_copy(row_v.at[pl.ds(D, D)], amax_hbm.at[pl.ds(gout, D)])

    @pl.when(s < e)
    def _():
        kh = s // CHUNK
        kt = (e + CHUNK - 1) // CHUNK
        zerov = jnp.zeros((16,), jnp.float32)
        ninfv = jnp.full((16,), -jnp.inf, jnp.float32)
        acc0 = tuple([zerov] * NCOL + [ninfv] * NCOL)

        pltpu.async_copy(
            x_hbm.at[pl.ds(pl.multiple_of(kh * CHUNK, 8), CHUNK)],
            chunk_v.at[pl.ds(0, CHUNK)], sem)

        def chunk_body(k, acc):
            par = (k - kh) & 1
            buf_row0 = pl.multiple_of(par * CHUNK, 8)
            pltpu.make_async_copy(
                x_hbm.at[pl.ds(0, CHUNK)],
                chunk_v.at[pl.ds(buf_row0, CHUNK)], sem).wait()

            @pl.when(k + 1 < kt)
            def _():
                nbuf = pl.multiple_of(((k + 1 - kh) & 1) * CHUNK, 8)
                pltpu.async_copy(
                    x_hbm.at[pl.ds(pl.multiple_of((k + 1) * CHUNK, 8), CHUNK)],
                    chunk_v.at[pl.ds(nbuf, CHUNK)], sem)

            base = k * CHUNK
            lo = jnp.maximum(s, base) - base
            hi = jnp.minimum(e, base + CHUNK) - base

            def group_body(grp, a):
                return _accum_group(chunk_v, buf_row0, grp * GROUP, lo, hi,
                                    a, masked=True)

            return lax.fori_loop(0, NGROUPS, group_body, acc)

        acc = lax.fori_loop(kh, kt, chunk_body, acc0)

        cntv = jnp.full((16,), (e - s).astype(jnp.float32))
        inv = 1.0 / cntv
        for c in range(NCOL):
            row_v[pl.ds(c * 16, 16)] = acc[c] * inv
            row_v[pl.ds(D + c * 16, 16)] = acc[NCOL + c]
        pltpu.sync_copy(row_v.at[pl.ds(0, D)], mean_hbm.at[pl.ds(gout, D)])
        pltpu.sync_copy(row_v.at[pl.ds(D, D)], amax_hbm.at[pl.ds(gout, D)])


def _seg_reduce_body(x_hbm, batch_hbm, mean_hbm, amax_hbm, batch_v, chunk_v,
                     row_v, sem):
    wid = lax.axis_index("s") * NCORES + lax.axis_index("c")
    g0 = 2 * wid
    pltpu.sync_copy(batch_hbm, batch_v)
    s0 = _lower_bound(batch_v, g0)
    s1 = _lower_bound(batch_v, g0 + 1)
    s2 = _lower_bound(batch_v, g0 + 2)

    def graph_body(i, carry):
        g = g0 + i
        s = jnp.where(i == 0, s0, s1)
        e = jnp.where(i == 0, s1, s2)
        _process_graph(x_hbm, chunk_v, row_v, mean_hbm, amax_hbm, sem, g, s, e)
        return carry

    lax.fori_loop(0, 2, graph_body, jnp.int32(0))


_SC_KERNEL_KWARGS = dict(
    out_type=(jax.ShapeDtypeStruct((B * D,), jnp.float32),
              jax.ShapeDtypeStruct((B * D,), jnp.float32)),
    mesh=plsc.VectorSubcoreMesh(core_axis_name="c", subcore_axis_name="s"),
    scratch_types=[
        pltpu.VMEM((N,), jnp.int32),
        pltpu.VMEM((2 * CHUNK, D), jnp.float32),
        pltpu.VMEM((2 * D,), jnp.float32),
        pltpu.SemaphoreType.DMA,
    ],
    compiler_params=pltpu.CompilerParams(needs_layout_passes=False),
)

_seg_reduce = pl.kernel(_seg_reduce_body, **_SC_KERNEL_KWARGS)


def _mlp_body(u_ref, mean_ref, amax_ref, w1_ref, b1_ref, w2_ref, b2_ref,
              w3_ref, b3_ref, o_ref):
    h = jnp.concatenate([u_ref[...], mean_ref[...], amax_ref[...]], axis=1)
    h = jnp.dot(h, w1_ref[...], preferred_element_type=jnp.float32,
                precision=lax.Precision.HIGHEST) + b1_ref[...]
    h = jnp.maximum(h, 0.0)
    h = jnp.dot(h, w2_ref[...], preferred_element_type=jnp.float32,
                precision=lax.Precision.HIGHEST) + b2_ref[...]
    h = jnp.maximum(h, 0.0)
    o_ref[...] = jnp.dot(h, w3_ref[...], preferred_element_type=jnp.float32,
                         precision=lax.Precision.HIGHEST) + b3_ref[...]


def kernel(x, edge_index, edge_attr, u, batch, W1, b1, W2, b2, W3, b3):
    del edge_index, edge_attr
    batch32 = batch.astype(jnp.int32)
    mean_flat, amax_flat = _seg_reduce(x, batch32)
    mean = mean_flat.reshape(B, D)
    amax = amax_flat.reshape(B, D)
    out = pl.pallas_call(
        _mlp_body,
        out_shape=jax.ShapeDtypeStruct((B, W3.shape[1]), jnp.float32),
    )(u, mean, amax, W1, b1.reshape(1, -1), W2, b2.reshape(1, -1),
      W3, b3.reshape(1, -1))
    return out


# speedup vs baseline: 1.2928x; 1.2928x over previous
import jax
import jax.numpy as jnp
from jax import lax
from jax.experimental import pallas as pl
from jax.experimental.pallas import tpu as pltpu
from jax.experimental.pallas import tpu_sc as plsc

N = 50000
D = 256
B = 64
CHUNK = 80
GROUP = 8
NGROUPS = CHUNK // GROUP
NCOL = D // 16
NCORES = 2
NSUBCORES = 16


def _lower_bound(batch_ref, t):
    def step(_, c):
        lo, hi = c
        mid = (lo + hi) // 2
        idx = jnp.full((16,), mid, jnp.int32)
        val = jnp.min(plsc.load_gather(batch_ref, [idx]))
        go_right = val < t
        live = lo < hi
        new_lo = jnp.where(live & go_right, mid + 1, lo)
        new_hi = jnp.where(live & ~go_right, mid, hi)
        return (new_lo, new_hi)

    lo, _ = lax.fori_loop(0, 16, step, (jnp.int32(0), jnp.int32(N)))
    return lo


def _tree_sum(vals):
    while len(vals) > 1:
        vals = [vals[i] + vals[i + 1] for i in range(0, len(vals) - 1, 2)] + (
            [vals[-1]] if len(vals) % 2 else [])
    return vals[0]


def _tree_max(vals):
    while len(vals) > 1:
        vals = [jnp.maximum(vals[i], vals[i + 1])
                for i in range(0, len(vals) - 1, 2)] + (
            [vals[-1]] if len(vals) % 2 else [])
    return vals[0]


def _accum_group(chunk_v, buf_row0, r0, lo, hi, acc, masked):
    zerov = jnp.zeros((16,), jnp.float32)
    ninfv = jnp.full((16,), -jnp.inf, jnp.float32)
    if masked:
        vvs = [jnp.full((16,), ((r0 + u) >= lo) & ((r0 + u) < hi))
               for u in range(GROUP)]
    new = list(acc)
    for c in range(NCOL):
        xv = [chunk_v[buf_row0 + r0 + u, pl.ds(c * 16, 16)]
              for u in range(GROUP)]
        if masked:
            xs = [jnp.where(vvs[u], xv[u], zerov) for u in range(GROUP)]
            xm = [jnp.where(vvs[u], xv[u], ninfv) for u in range(GROUP)]
        else:
            xs = xv
            xm = xv
        new[c] = new[c] + _tree_sum(xs)
        new[NCOL + c] = jnp.maximum(new[NCOL + c], _tree_max(xm))
    return tuple(new)


def _process_graph(x_hbm, chunk_v, row_v, mean_hbm, amax_hbm, sem, g, s, e):
    gout = pl.multiple_of(g * D, 8)

    @pl.when(s >= e)
    def _():
        zerov = jnp.zeros((16,), jnp.float32)
        for c in range(NCOL):
            row_v[pl.ds(c * 16, 16)] = zerov
            row_v[pl.ds(D + c * 16, 16)] = zerov
        pltpu.sync_copy(row_v.at[pl.ds(0, D)], mean_hbm.at[pl.ds(gout, D)])
        pltpu.sync_copy(row_v.at[pl.ds(D, D)], amax_hbm.at[pl.ds(gout, D)])

    @pl.when(s < e)
    def _():
        kh = s // CHUNK
        kt = (e + CHUNK - 1) // CHUNK
        zerov = jnp.zeros((16,), jnp.float32)
        ninfv = jnp.full((16,), -jnp.inf, jnp.float32)
        acc0 = tuple([zerov] * NCOL + [ninfv] * NCOL)

        pltpu.async_copy(
            x_hbm.at[pl.ds(pl.multiple_of(kh * CHUNK, 8), CHUNK)],
            chunk_v.at[pl.ds(0, CHUNK)], sem)

        def chunk_body(k, acc):
            par = (k - kh) & 1
            buf_row0 = pl.multiple_of(par * CHUNK, 8)
            pltpu.make_async_copy(
                x_hbm.at[pl.ds(0, CHUNK)],
                chunk_v.at[pl.ds(buf_row0, CHUNK)], sem).wait()

            @pl.when(k + 1 < kt)
            def _():
                nbuf = pl.multiple_of(((k + 1 - kh) & 1) * CHUNK, 8)
                pltpu.async_copy(
                    x_hbm.at[pl.ds(pl.multiple_of((k + 1) * CHUNK, 8), CHUNK)],
                    chunk_v.at[pl.ds(nbuf, CHUNK)], sem)

            base = k * CHUNK
            lo = jnp.maximum(s, base) - base
            hi = jnp.minimum(e, base + CHUNK) - base

            def group_body(grp, a):
                r0 = grp * GROUP
                inside = (r0 >= lo) & ((r0 + GROUP) <= hi)

                def fast(aa):
                    return _accum_group(chunk_v, buf_row0, r0, lo, hi, aa,
                                        masked=False)

                def slow(aa):
                    return _accum_group(chunk_v, buf_row0, r0, lo, hi, aa,
                                        masked=True)

                return lax.cond(inside, fast, slow, a)

            return lax.fori_loop(0, NGROUPS, group_body, acc)

        acc = lax.fori_loop(kh, kt, chunk_body, acc0)

        cntv = jnp.full((16,), (e - s).astype(jnp.float32))
        inv = 1.0 / cntv
        for c in range(NCOL):
            row_v[pl.ds(c * 16, 16)] = acc[c] * inv
            row_v[pl.ds(D + c * 16, 16)] = acc[NCOL + c]
        pltpu.sync_copy(row_v.at[pl.ds(0, D)], mean_hbm.at[pl.ds(gout, D)])
        pltpu.sync_copy(row_v.at[pl.ds(D, D)], amax_hbm.at[pl.ds(gout, D)])


def _seg_reduce_body(x_hbm, batch_hbm, mean_hbm, amax_hbm, batch_v, chunk_v,
                     row_v, sem):
    wid = lax.axis_index("s") * NCORES + lax.axis_index("c")
    g0 = 2 * wid
    pltpu.sync_copy(batch_hbm, batch_v)
    s0 = _lower_bound(batch_v, g0)
    s1 = _lower_bound(batch_v, g0 + 1)
    s2 = _lower_bound(batch_v, g0 + 2)

    def graph_body(i, carry):
        g = g0 + i
        s = jnp.where(i == 0, s0, s1)
        e = jnp.where(i == 0, s1, s2)
        _process_graph(x_hbm, chunk_v, row_v, mean_hbm, amax_hbm, sem, g, s, e)
        return carry

    lax.fori_loop(0, 2, graph_body, jnp.int32(0))


_SC_KERNEL_KWARGS = dict(
    out_type=(jax.ShapeDtypeStruct((B * D,), jnp.float32),
              jax.ShapeDtypeStruct((B * D,), jnp.float32)),
    mesh=plsc.VectorSubcoreMesh(core_axis_name="c", subcore_axis_name="s"),
    scratch_types=[
        pltpu.VMEM((N,), jnp.int32),
        pltpu.VMEM((2 * CHUNK, D), jnp.float32),
        pltpu.VMEM((2 * D,), jnp.float32),
        pltpu.SemaphoreType.DMA,
    ],
    compiler_params=pltpu.CompilerParams(needs_layout_passes=False),
)

_seg_reduce = pl.kernel(_seg_reduce_body, **_SC_KERNEL_KWARGS)


def _mlp_body(u_ref, mean_ref, amax_ref, w1_ref, b1_ref, w2_ref, b2_ref,
              w3_ref, b3_ref, o_ref):
    mean2 = mean_ref[...].reshape(B, D)
    amax2 = amax_ref[...].reshape(B, D)
    h = jnp.concatenate([u_ref[...], mean2, amax2], axis=1)
    h = jnp.dot(h, w1_ref[...], preferred_element_type=jnp.float32,
                precision=lax.Precision.HIGHEST) + b1_ref[...]
    h = jnp.maximum(h, 0.0)
    h = jnp.dot(h, w2_ref[...], preferred_element_type=jnp.float32,
                precision=lax.Precision.HIGHEST) + b2_ref[...]
    h = jnp.maximum(h, 0.0)
    o_ref[...] = jnp.dot(h, w3_ref[...], preferred_element_type=jnp.float32,
                         precision=lax.Precision.HIGHEST) + b3_ref[...]


def kernel(x, edge_index, edge_attr, u, batch, W1, b1, W2, b2, W3, b3):
    del edge_index, edge_attr
    batch32 = batch.astype(jnp.int32)
    mean_flat, amax_flat = _seg_reduce(x, batch32)
    out = pl.pallas_call(
        _mlp_body,
        out_shape=jax.ShapeDtypeStruct((B, W3.shape[1]), jnp.float32),
    )(u, mean_flat, amax_flat, W1, b1.reshape(1, -1), W2, b2.reshape(1, -1),
      W3, b3.reshape(1, -1))
    return out
